# Initial kernel scaffold; baseline (speedup 1.0000x reference)
import functools

import jax
import jax.numpy as jnp
from jax import lax
from jax.experimental import pallas as pl
from jax.experimental.pallas import tpu as pltpu
from jax.experimental.pallas import tpu_sc as plsc

NUM_F = 2344
ACC = 32
A_SLOTS = 32
B_TOTAL = 16384
NC, NS, L = 2, 16, 16
NW = NC * NS
BPW = B_TOTAL // NW
NG = BPW // L


def _fq(x, scale, qmin, qmax):
    s = 1.0 / scale
    return jnp.clip(jnp.round(x / s), qmin, qmax) * s


def _splat(params_ref, off):
    return plsc.load_gather(params_ref, [jnp.full((L,), off, jnp.int32)])


def _nnue_body(table_hbm, bf_hbm, wf_hbm, stm_hbm, params_hbm, out_hbm,
               table_v, bf_v, wf_v, stm_v, params_v, out_v, accb_v, accw_v):
    wid = lax.axis_index("s") * NC + lax.axis_index("c")

    pltpu.sync_copy(table_hbm, table_v)
    pltpu.sync_copy(bf_hbm.at[wid], bf_v)
    pltpu.sync_copy(wf_hbm.at[wid], wf_v)
    pltpu.sync_copy(stm_hbm.at[wid], stm_v)
    pltpu.sync_copy(params_hbm, params_v)

    def group_body(g, _):
        for c in range(ACC):
            b = _splat(params_v, 64 + c)
            accb_v[c, :] = b
            accw_v[c, :] = b

        def slot_body(a, _):
            idxb = bf_v[a, pl.ds(g * L, L)]
            idxw = wf_v[a, pl.ds(g * L, L)]
            for c in range(ACC):
                row = jnp.full((L,), c, jnp.int32)
                plsc.addupdate(accb_v.at[c], plsc.load_gather(table_v, [row, idxb]))
                plsc.addupdate(accw_v.at[c], plsc.load_gather(table_v, [row, idxw]))
            return 0

        lax.fori_loop(0, A_SLOTS, slot_body, 0)

        b_first = stm_v[pl.ds(g * L, L)] == 0
        out = _splat(params_v, 96)
        for c in range(ACC):
            hb = jnp.clip(accb_v[c, :], 0.0, 1.0)
            hw = jnp.clip(accw_v[c, :], 0.0, 1.0)
            first = jnp.where(b_first, hb, hw)
            second = jnp.where(b_first, hw, hb)
            out = out + first * _splat(params_v, c) + second * _splat(params_v, 32 + c)
        out_v[pl.ds(g * L, L)] = out
        return 0

    lax.fori_loop(0, NG, group_body, 0)
    pltpu.sync_copy(out_v, out_hbm.at[wid])


def kernel(black_features, white_features, stm, ft_weight, ft_bias,
           l1_weight, l1_bias):
    table_t = _fq(ft_weight, 127, -32768, 32767).T
    ftb_q = _fq(ft_bias, 127, -32768, 32767)
    l2w_q = _fq(l1_weight, 64, -128, 127).reshape(-1)
    params = jnp.concatenate(
        [l2w_q, ftb_q, l1_bias, jnp.zeros((31,), jnp.float32)])

    def to_tiles(f):
        return f.T.reshape(A_SLOTS, NW, BPW).transpose(1, 0, 2)

    bf = to_tiles(black_features)
    wf = to_tiles(white_features)
    stm_t = stm.reshape(NW, BPW)

    mesh = plsc.VectorSubcoreMesh(core_axis_name="c", subcore_axis_name="s")
    out = pl.kernel(
        _nnue_body,
        out_type=jax.ShapeDtypeStruct((NW, BPW), jnp.float32),
        mesh=mesh,
        scratch_types=[
            pltpu.VMEM((ACC, NUM_F), jnp.float32),
            pltpu.VMEM((A_SLOTS, BPW), jnp.int32),
            pltpu.VMEM((A_SLOTS, BPW), jnp.int32),
            pltpu.VMEM((BPW,), jnp.int32),
            pltpu.VMEM((128,), jnp.float32),
            pltpu.VMEM((BPW,), jnp.float32),
            pltpu.VMEM((ACC, L), jnp.float32),
            pltpu.VMEM((ACC, L), jnp.float32),
        ],
    )(table_t, bf, wf, stm_t, params)
    return out.reshape(B_TOTAL, 1)


# speedup vs baseline: 40.3804x; 40.3804x over previous
import functools

import jax
import jax.numpy as jnp
from jax import lax
from jax.experimental import pallas as pl
from jax.experimental.pallas import tpu as pltpu
from jax.experimental.pallas import tpu_sc as plsc

NUM_F = 2344
ACC = 32
A_SLOTS = 32
B_TOTAL = 16384
NC, NS, L = 2, 16, 16
NW = NC * NS
BPW = B_TOTAL // NW
NG = BPW // L


def _fq(x, scale, qmin, qmax):
    s = 1.0 / scale
    return jnp.clip(jnp.round(x / s), qmin, qmax) * s


def _splat(params_ref, off):
    return plsc.load_gather(params_ref, [jnp.full((L,), off, jnp.int32)])


def _nnue_body(table_hbm, bf_hbm, wf_hbm, stm_hbm, params_hbm, out_hbm,
               table_v, bf_v, wf_v, stm_v, params_v, out_v, accb_v, accw_v):
    wid = lax.axis_index("s") * NC + lax.axis_index("c")

    pltpu.sync_copy(table_hbm, table_v)
    pltpu.sync_copy(bf_hbm.at[wid], bf_v)
    pltpu.sync_copy(wf_hbm.at[wid], wf_v)
    pltpu.sync_copy(stm_hbm.at[wid], stm_v)
    pltpu.sync_copy(params_hbm, params_v)

    CH = 16

    def group_body(g, _):
        for idx_v, acc_v in ((bf_v, accb_v), (wf_v, accw_v)):
            for half in range(ACC // CH):
                def slot_body(a, accs, idx_v=idx_v, half=half):
                    idx = idx_v[a, pl.ds(g * L, L)]
                    return tuple(
                        acc + plsc.load_gather(
                            table_v, [jnp.full((L,), half * CH + c, jnp.int32), idx])
                        for c, acc in enumerate(accs))

                init = tuple(_splat(params_v, 65 + half * CH + c) for c in range(CH))
                accs = lax.fori_loop(0, A_SLOTS, slot_body, init)
                for c in range(CH):
                    acc_v[half * CH + c, :] = accs[c]

        b_first = stm_v[pl.ds(g * L, L)] == 0
        out = _splat(params_v, 97)
        for c in range(ACC):
            hb = jnp.clip(accb_v[c, :], 0.0, 1.0)
            hw = jnp.clip(accw_v[c, :], 0.0, 1.0)
            first = jnp.where(b_first, hb, hw)
            second = jnp.where(b_first, hw, hb)
            out = out + first * _splat(params_v, 1 + c) + second * _splat(params_v, 33 + c)
        out_v[pl.ds(g * L, L)] = out
        return 0

    lax.fori_loop(0, NG, group_body, 0)
    pltpu.sync_copy(out_v, out_hbm.at[wid])


def kernel(black_features, white_features, stm, ft_weight, ft_bias,
           l1_weight, l1_bias):
    table_t = _fq(ft_weight, 127, -32768, 32767).T
    ftb_q = _fq(ft_bias, 127, -32768, 32767)
    l2w_q = _fq(l1_weight, 64, -128, 127).reshape(-1)
    params = jnp.concatenate(
        [jnp.zeros((1,), jnp.float32), l2w_q, ftb_q, l1_bias,
         jnp.zeros((30,), jnp.float32)])

    def to_tiles(f):
        return f.T.reshape(A_SLOTS, NW, BPW).transpose(1, 0, 2)

    bf = to_tiles(black_features)
    wf = to_tiles(white_features)
    stm_t = stm.reshape(NW, BPW)

    mesh = plsc.VectorSubcoreMesh(core_axis_name="c", subcore_axis_name="s")
    out = pl.kernel(
        _nnue_body,
        out_type=jax.ShapeDtypeStruct((NW, BPW), jnp.float32),
        mesh=mesh,
        compiler_params=pltpu.CompilerParams(
            needs_layout_passes=False, use_tc_tiling_on_sc=False),
        scratch_types=[
            pltpu.VMEM((ACC, NUM_F), jnp.float32),
            pltpu.VMEM((A_SLOTS, BPW), jnp.int32),
            pltpu.VMEM((A_SLOTS, BPW), jnp.int32),
            pltpu.VMEM((BPW,), jnp.int32),
            pltpu.VMEM((128,), jnp.float32),
            pltpu.VMEM((BPW,), jnp.float32),
            pltpu.VMEM((ACC, L), jnp.float32),
            pltpu.VMEM((ACC, L), jnp.float32),
        ],
    )(table_t, bf, wf, stm_t, params)
    return out.reshape(B_TOTAL, 1)
